# Initial kernel scaffold; baseline (speedup 1.0000x reference)
import jax
import jax.numpy as jnp
from jax.experimental import pallas as pl


def kernel(z, labels, sample_weights, Wa, ba, Wb, bb, epoch, total_epochs):
    raise NotImplementedError("write your pallas kernel here")


# speedup vs baseline: 4.1563x; 4.1563x over previous
import jax
import jax.numpy as jnp
from jax import lax
from jax.experimental import pallas as pl
from jax.experimental.pallas import tpu as pltpu

_B = 16384
_D = 128
_K = 11468
_ROWS = 2048
_NBLK = _B // _ROWS


def _focal_from_margin(m):
    ce = jnp.maximum(m, 0.0) + jnp.log1p(jnp.exp(-jnp.abs(m)))
    pt = jnp.exp(-ce)
    return 0.5 * (1.0 - pt) ** 2 * ce


def _loss_body(db_ref, dw_ref, z_ref, lab_ref, w_ref,
               la_ref, lb_ref, pa_ref, pb_ref):
    z = z_ref[...]
    ua = jnp.sum(z * dw_ref[0:1, :], axis=1) + db_ref[0]
    ub = jnp.sum(z * dw_ref[1:2, :], axis=1) + db_ref[1]
    sgn = 1.0 - 2.0 * lab_ref[...]
    la = _focal_from_margin(ua * sgn)
    lb = _focal_from_margin(ub * sgn)
    w = w_ref[...]
    la_ref[...] = la
    lb_ref[...] = lb
    pa_ref[...] = la * w
    pb_ref[...] = lb * w


def _select_threshold(keys):
    P = jnp.int32(0)
    t = jnp.float32(_K)
    for b in range(30, -1, -1):
        c0 = jnp.sum(((keys >> b) == (P >> b)).astype(jnp.float32))
        go1 = t > c0
        t = jnp.where(go1, t - c0, t)
        P = jnp.where(go1, P | (1 << b), P)
    return P, t


def _masked_sum(keys, vals, K, T, ut, lt):
    lt_mask = (keys < K).astype(jnp.float32)
    s_lt = jnp.sum(vals * lt_mask)
    eq = (keys == K).astype(jnp.float32)
    inner = jnp.dot(eq, ut, preferred_element_type=jnp.float32)
    rowtot = inner[:, 127:128]
    rowoff = jnp.dot(lt, rowtot, preferred_element_type=jnp.float32)
    g = inner + rowoff
    incl = eq * (g <= T).astype(jnp.float32)
    return s_lt + jnp.sum(vals * incl)


def _select_body(la_ref, lb_ref, pa_ref, pb_ref,
                 sca_ref, scb_ref, spa_ref, spb_ref):
    la = la_ref[...]
    lb = lb_ref[...]
    pa = pa_ref[...]
    pb = pb_ref[...]
    ka = lax.bitcast_convert_type(la, jnp.int32)
    kb = lax.bitcast_convert_type(lb, jnp.int32)
    r = lax.broadcasted_iota(jnp.int32, (128, 128), 0)
    c = lax.broadcasted_iota(jnp.int32, (128, 128), 1)
    ut = (r <= c).astype(jnp.float32)
    ltm = (c < r).astype(jnp.float32)
    Ka, Ta = _select_threshold(ka)
    Kb, Tb = _select_threshold(kb)
    scb_ref[0, 0] = _masked_sum(ka, pb, Ka, Ta, ut, ltm)
    sca_ref[0, 0] = _masked_sum(kb, pa, Kb, Tb, ut, ltm)
    spa_ref[0, 0] = jnp.sum(pa)
    spb_ref[0, 0] = jnp.sum(pb)


def kernel(z, labels, sample_weights, Wa, ba, Wb, bb, epoch, total_epochs):
    dw = jnp.stack([Wa[:, 1] - Wa[:, 0], Wb[:, 1] - Wb[:, 0]])
    db = jnp.stack([ba[1] - ba[0], bb[1] - bb[0]])
    labf = labels.astype(jnp.float32)

    outs = pl.pallas_call(
        _loss_body,
        grid=(_NBLK,),
        in_specs=[
            pl.BlockSpec(memory_space=pltpu.SMEM),
            pl.BlockSpec((2, _D), lambda i: (0, 0)),
            pl.BlockSpec((_ROWS, _D), lambda i: (i, 0)),
            pl.BlockSpec((_ROWS,), lambda i: (i,)),
            pl.BlockSpec((_ROWS,), lambda i: (i,)),
        ],
        out_specs=[pl.BlockSpec((_ROWS,), lambda i: (i,))] * 4,
        out_shape=[jax.ShapeDtypeStruct((_B,), jnp.float32)] * 4,
    )(db, dw, z, labf, sample_weights)
    la, lb, pa, pb = outs

    shp = (_B // 128, 128)
    sca, scb, spa, spb = pl.pallas_call(
        _select_body,
        in_specs=[pl.BlockSpec(shp, lambda: (0, 0))] * 4,
        out_specs=[pl.BlockSpec(memory_space=pltpu.SMEM)] * 4,
        out_shape=[jax.ShapeDtypeStruct((1, 1), jnp.float32)] * 4,
    )(la.reshape(shp), lb.reshape(shp), pa.reshape(shp), pb.reshape(shp))

    use_ct = epoch >= 10
    sup_a = jnp.where(use_ct, sca[0, 0] / _K, spa[0, 0] / _B)
    sup_b = jnp.where(use_ct, scb[0, 0] / _K, spb[0, 0] / _B)
    return 0.5 * (sup_a + sup_b)


# speedup vs baseline: 7.0496x; 1.6961x over previous
import jax
import jax.numpy as jnp
from jax import lax
from jax.experimental import pallas as pl
from jax.experimental.pallas import tpu as pltpu

_B = 16384
_D = 128
_K = 11468
_NBLK = 8
_SUB = 16


def _focal_from_margin(m):
    ce = jnp.maximum(m, 0.0) + jnp.log1p(jnp.exp(-jnp.abs(m)))
    pt = jnp.exp(-ce)
    return 0.5 * (1.0 - pt) ** 2 * ce


def _select_thresholds(ka, kb):
    Pa = jnp.int32(0)
    Pb = jnp.int32(0)
    ta = jnp.float32(_K)
    tb = jnp.float32(_K)
    for b in range(30, -1, -1):
        c0a = jnp.sum(((ka >> b) == (Pa >> b)).astype(jnp.float32))
        c0b = jnp.sum(((kb >> b) == (Pb >> b)).astype(jnp.float32))
        ga = ta > c0a
        gb = tb > c0b
        ta = jnp.where(ga, ta - c0a, ta)
        Pa = jnp.where(ga, Pa | (1 << b), Pa)
        tb = jnp.where(gb, tb - c0b, tb)
        Pb = jnp.where(gb, Pb | (1 << b), Pb)
    return Pa, ta, Pb, tb


def _masked_sum(keys, vals, K, T, ut, lt):
    s_lt = jnp.sum(vals * (keys < K).astype(jnp.float32))
    eq = (keys == K).astype(jnp.float32)
    inner = jnp.dot(eq, ut, preferred_element_type=jnp.float32)
    rowtot = inner[:, 127:128]
    rowoff = jnp.dot(lt, rowtot, preferred_element_type=jnp.float32)
    g = inner + rowoff
    incl = eq * (g <= T).astype(jnp.float32)
    return s_lt + jnp.sum(vals * incl)


def _body(b_ref, ep_ref, waT_ref, wbT_ref, z_ref, lab_ref, w_ref, out_ref,
          la_s, lb_s, pa_s, pb_s):
    i = pl.program_id(0)

    @pl.when(i < _NBLK)
    def _compute_losses():
        z = z_ref[...]
        dwa = waT_ref[1:2, :] - waT_ref[0:1, :]
        dwb = wbT_ref[1:2, :] - wbT_ref[0:1, :]
        ua = jnp.sum(z * dwa[None], axis=2) + b_ref[0]
        ub = jnp.sum(z * dwb[None], axis=2) + b_ref[1]
        sgn = 1.0 - 2.0 * lab_ref[...]
        la = _focal_from_margin(ua * sgn)
        lb = _focal_from_margin(ub * sgn)
        w = w_ref[...]
        r0 = i * _SUB
        la_s[pl.ds(r0, _SUB), :] = la
        lb_s[pl.ds(r0, _SUB), :] = lb
        pa_s[pl.ds(r0, _SUB), :] = la * w
        pb_s[pl.ds(r0, _SUB), :] = lb * w

    @pl.when(i == _NBLK)
    def _select_and_reduce():
        la = la_s[...]
        lb = lb_s[...]
        pa = pa_s[...]
        pb = pb_s[...]
        ka = lax.bitcast_convert_type(la, jnp.int32)
        kb = lax.bitcast_convert_type(lb, jnp.int32)
        r = lax.broadcasted_iota(jnp.int32, (128, 128), 0)
        c = lax.broadcasted_iota(jnp.int32, (128, 128), 1)
        ut = (r <= c).astype(jnp.float32)
        ltm = (c < r).astype(jnp.float32)
        Ka, Ta, Kb, Tb = _select_thresholds(ka, kb)
        scb = _masked_sum(ka, pb, Ka, Ta, ut, ltm)
        sca = _masked_sum(kb, pa, Kb, Tb, ut, ltm)
        spa = jnp.sum(pa)
        spb = jnp.sum(pb)
        use_ct = ep_ref[0] >= 10
        sup_a = jnp.where(use_ct, sca / _K, spa / _B)
        sup_b = jnp.where(use_ct, scb / _K, spb / _B)
        out_ref[0, 0] = 0.5 * (sup_a + sup_b)


def kernel(z, labels, sample_weights, Wa, ba, Wb, bb, epoch, total_epochs):
    bvec = jnp.stack([ba[1] - ba[0], bb[1] - bb[0]])
    epv = jnp.asarray(epoch, jnp.int32).reshape(1)
    z3 = z.reshape(128, 128, 128)
    lab128 = labels.astype(jnp.float32).reshape(128, 128)
    w128 = sample_weights.reshape(128, 128)

    last = _NBLK - 1
    out = pl.pallas_call(
        _body,
        grid=(_NBLK + 1,),
        in_specs=[
            pl.BlockSpec(memory_space=pltpu.SMEM),
            pl.BlockSpec(memory_space=pltpu.SMEM),
            pl.BlockSpec((2, _D), lambda i: (0, 0)),
            pl.BlockSpec((2, _D), lambda i: (0, 0)),
            pl.BlockSpec((_SUB, 128, 128), lambda i: (jnp.minimum(i, last), 0, 0)),
            pl.BlockSpec((_SUB, 128), lambda i: (jnp.minimum(i, last), 0)),
            pl.BlockSpec((_SUB, 128), lambda i: (jnp.minimum(i, last), 0)),
        ],
        out_specs=pl.BlockSpec(memory_space=pltpu.SMEM),
        out_shape=jax.ShapeDtypeStruct((1, 1), jnp.float32),
        scratch_shapes=[pltpu.VMEM((128, 128), jnp.float32)] * 4,
    )(bvec, epv, Wa.T, Wb.T, z3, lab128, w128)
    return out[0, 0]


# speedup vs baseline: 7.0683x; 1.0027x over previous
import jax
import jax.numpy as jnp
from jax import lax
from jax.experimental import pallas as pl
from jax.experimental.pallas import tpu as pltpu

_B = 16384
_D = 128
_K = 11468
_NBLK = 4
_SUB = 32


def _focal_from_margin(m):
    ce = jnp.maximum(m, 0.0) + jnp.log1p(jnp.exp(-jnp.abs(m)))
    pt = jnp.exp(-ce)
    return 0.5 * (1.0 - pt) ** 2 * ce


def _select_thresholds(ka, kb):
    Pa = jnp.int32(0)
    Pb = jnp.int32(0)
    ta = jnp.float32(_K)
    tb = jnp.float32(_K)
    for b in range(30, -1, -1):
        c0a = jnp.sum(((ka >> b) == (Pa >> b)).astype(jnp.float32))
        c0b = jnp.sum(((kb >> b) == (Pb >> b)).astype(jnp.float32))
        ga = ta > c0a
        gb = tb > c0b
        ta = jnp.where(ga, ta - c0a, ta)
        Pa = jnp.where(ga, Pa | (1 << b), Pa)
        tb = jnp.where(gb, tb - c0b, tb)
        Pb = jnp.where(gb, Pb | (1 << b), Pb)
    return Pa, ta, Pb, tb


def _masked_sum(keys, vals, K, T, ut, lt):
    s_lt = jnp.sum(vals * (keys < K).astype(jnp.float32))
    eq = (keys == K).astype(jnp.float32)
    inner = jnp.dot(eq, ut, preferred_element_type=jnp.float32)
    rowtot = inner[:, 127:128]
    rowoff = jnp.dot(lt, rowtot, preferred_element_type=jnp.float32)
    g = inner + rowoff
    incl = eq * (g <= T).astype(jnp.float32)
    return s_lt + jnp.sum(vals * incl)


def _body(b_ref, ep_ref, waT_ref, wbT_ref, z_ref, lab_ref, w_ref, out_ref,
          la_s, lb_s, pa_s, pb_s):
    i = pl.program_id(0)

    @pl.when(i < _NBLK)
    def _compute_losses():
        z = z_ref[...]
        dwa = waT_ref[1:2, :] - waT_ref[0:1, :]
        dwb = wbT_ref[1:2, :] - wbT_ref[0:1, :]
        ua = jnp.sum(z * dwa[None], axis=2) + b_ref[0]
        ub = jnp.sum(z * dwb[None], axis=2) + b_ref[1]
        sgn = 1.0 - 2.0 * lab_ref[...]
        la = _focal_from_margin(ua * sgn)
        lb = _focal_from_margin(ub * sgn)
        w = w_ref[...]
        r0 = i * _SUB
        la_s[pl.ds(r0, _SUB), :] = la
        lb_s[pl.ds(r0, _SUB), :] = lb
        pa_s[pl.ds(r0, _SUB), :] = la * w
        pb_s[pl.ds(r0, _SUB), :] = lb * w

    @pl.when(i == _NBLK)
    def _select_and_reduce():
        la = la_s[...]
        lb = lb_s[...]
        pa = pa_s[...]
        pb = pb_s[...]
        ka = lax.bitcast_convert_type(la, jnp.int32)
        kb = lax.bitcast_convert_type(lb, jnp.int32)
        r = lax.broadcasted_iota(jnp.int32, (128, 128), 0)
        c = lax.broadcasted_iota(jnp.int32, (128, 128), 1)
        ut = (r <= c).astype(jnp.float32)
        ltm = (c < r).astype(jnp.float32)
        Ka, Ta, Kb, Tb = _select_thresholds(ka, kb)
        scb = _masked_sum(ka, pb, Ka, Ta, ut, ltm)
        sca = _masked_sum(kb, pa, Kb, Tb, ut, ltm)
        spa = jnp.sum(pa)
        spb = jnp.sum(pb)
        use_ct = ep_ref[0] >= 10
        sup_a = jnp.where(use_ct, sca / _K, spa / _B)
        sup_b = jnp.where(use_ct, scb / _K, spb / _B)
        out_ref[0, 0] = 0.5 * (sup_a + sup_b)


def kernel(z, labels, sample_weights, Wa, ba, Wb, bb, epoch, total_epochs):
    bvec = jnp.stack([ba[1] - ba[0], bb[1] - bb[0]])
    epv = jnp.asarray(epoch, jnp.int32).reshape(1)
    z3 = z.reshape(128, 128, 128)
    lab128 = labels.astype(jnp.float32).reshape(128, 128)
    w128 = sample_weights.reshape(128, 128)

    last = _NBLK - 1
    out = pl.pallas_call(
        _body,
        grid=(_NBLK + 1,),
        in_specs=[
            pl.BlockSpec(memory_space=pltpu.SMEM),
            pl.BlockSpec(memory_space=pltpu.SMEM),
            pl.BlockSpec((2, _D), lambda i: (0, 0)),
            pl.BlockSpec((2, _D), lambda i: (0, 0)),
            pl.BlockSpec((_SUB, 128, 128), lambda i: (jnp.minimum(i, last), 0, 0)),
            pl.BlockSpec((_SUB, 128), lambda i: (jnp.minimum(i, last), 0)),
            pl.BlockSpec((_SUB, 128), lambda i: (jnp.minimum(i, last), 0)),
        ],
        out_specs=pl.BlockSpec(memory_space=pltpu.SMEM),
        out_shape=jax.ShapeDtypeStruct((1, 1), jnp.float32),
        scratch_shapes=[pltpu.VMEM((128, 128), jnp.float32)] * 4,
    )(bvec, epv, Wa.T, Wb.T, z3, lab128, w128)
    return out[0, 0]


# speedup vs baseline: 13.4566x; 1.9038x over previous
import jax
import jax.numpy as jnp
from jax import lax
from jax.experimental import pallas as pl
from jax.experimental.pallas import tpu as pltpu

_B = 16384
_D = 128
_K = 11468
_NBLK = 8
_ROWS = _B // _NBLK


def _focal_from_margin(m):
    ce = jnp.maximum(m, 0.0) + jnp.log1p(jnp.exp(-jnp.abs(m)))
    pt = jnp.exp(-ce)
    return 0.5 * (1.0 - pt) ** 2 * ce


def _select_thresholds(ka, kb):
    Pa = jnp.int32(0)
    Pb = jnp.int32(0)
    ta = jnp.float32(_K)
    tb = jnp.float32(_K)
    for b in range(30, -1, -1):
        c0a = jnp.sum(((ka >> b) == (Pa >> b)).astype(jnp.float32))
        c0b = jnp.sum(((kb >> b) == (Pb >> b)).astype(jnp.float32))
        ga = ta > c0a
        gb = tb > c0b
        ta = jnp.where(ga, ta - c0a, ta)
        Pa = jnp.where(ga, Pa | (1 << b), Pa)
        tb = jnp.where(gb, tb - c0b, tb)
        Pb = jnp.where(gb, Pb | (1 << b), Pb)
    return Pa, ta, Pb, tb


def _tie_index_cutoffs(eqa, eqb, Ta, Tb, idx):
    Pa = jnp.int32(0)
    Pb = jnp.int32(0)
    ta = Ta
    tb = Tb
    for b in range(13, -1, -1):
        ma = eqa & ((idx >> b) == (Pa >> b))
        mb = eqb & ((idx >> b) == (Pb >> b))
        c0a = jnp.sum(ma.astype(jnp.float32))
        c0b = jnp.sum(mb.astype(jnp.float32))
        ga = ta > c0a
        gb = tb > c0b
        ta = jnp.where(ga, ta - c0a, ta)
        Pa = jnp.where(ga, Pa | (1 << b), Pa)
        tb = jnp.where(gb, tb - c0b, tb)
        Pb = jnp.where(gb, Pb | (1 << b), Pb)
    return Pa, Pb


def _body(b_ref, ep_ref, waT_ref, wbT_ref, z_ref, lab_ref, w_ref, out_ref,
          la_s, lb_s, pa_s, pb_s):
    i = pl.program_id(0)

    @pl.when(i < _NBLK)
    def _compute_losses():
        z = z_ref[...]
        da = waT_ref[1:2, :] - waT_ref[0:1, :]
        db = wbT_ref[1:2, :] - wbT_ref[0:1, :]
        dvec = jnp.concatenate([da, db], axis=0)
        u = lax.dot_general(dvec, z, (((1,), (1,)), ((), ())),
                            preferred_element_type=jnp.float32)
        ua = u[0:1, :] + b_ref[0]
        ub = u[1:2, :] + b_ref[1]
        sgn = 1.0 - 2.0 * lab_ref[0]
        la = _focal_from_margin(ua * sgn)
        lb = _focal_from_margin(ub * sgn)
        w = w_ref[0]
        la_s[pl.ds(i, 1), :] = la
        lb_s[pl.ds(i, 1), :] = lb
        pa_s[pl.ds(i, 1), :] = la * w
        pb_s[pl.ds(i, 1), :] = lb * w

    @pl.when(i == _NBLK)
    def _select_and_reduce():
        la = la_s[...]
        lb = lb_s[...]
        pa = pa_s[...]
        pb = pb_s[...]
        ka = lax.bitcast_convert_type(la, jnp.int32)
        kb = lax.bitcast_convert_type(lb, jnp.int32)
        Ka, Ta, Kb, Tb = _select_thresholds(ka, kb)
        eqa = ka == Ka
        eqb = kb == Kb
        idx = (_ROWS * lax.broadcasted_iota(jnp.int32, (_NBLK, _ROWS), 0)
               + lax.broadcasted_iota(jnp.int32, (_NBLK, _ROWS), 1))
        Ia, Ib = _tie_index_cutoffs(eqa, eqb, Ta, Tb, idx)
        sela = (ka < Ka) | (eqa & (idx <= Ia))
        selb = (kb < Kb) | (eqb & (idx <= Ib))
        scb = jnp.sum(pb * sela.astype(jnp.float32))
        sca = jnp.sum(pa * selb.astype(jnp.float32))
        spa = jnp.sum(pa)
        spb = jnp.sum(pb)
        use_ct = ep_ref[0] >= 10
        sup_a = jnp.where(use_ct, sca / _K, spa / _B)
        sup_b = jnp.where(use_ct, scb / _K, spb / _B)
        out_ref[0, 0] = 0.5 * (sup_a + sup_b)


def kernel(z, labels, sample_weights, Wa, ba, Wb, bb, epoch, total_epochs):
    bvec = jnp.stack([ba[1] - ba[0], bb[1] - bb[0]])
    epv = jnp.asarray(epoch, jnp.int32).reshape(1)
    lab3 = labels.astype(jnp.float32).reshape(_NBLK, 1, _ROWS)
    w3 = sample_weights.reshape(_NBLK, 1, _ROWS)

    last = _NBLK - 1
    out = pl.pallas_call(
        _body,
        grid=(_NBLK + 1,),
        in_specs=[
            pl.BlockSpec(memory_space=pltpu.SMEM),
            pl.BlockSpec(memory_space=pltpu.SMEM),
            pl.BlockSpec((2, _D), lambda i: (0, 0)),
            pl.BlockSpec((2, _D), lambda i: (0, 0)),
            pl.BlockSpec((_ROWS, _D), lambda i: (jnp.minimum(i, last), 0)),
            pl.BlockSpec((1, 1, _ROWS), lambda i: (jnp.minimum(i, last), 0, 0)),
            pl.BlockSpec((1, 1, _ROWS), lambda i: (jnp.minimum(i, last), 0, 0)),
        ],
        out_specs=pl.BlockSpec(memory_space=pltpu.SMEM),
        out_shape=jax.ShapeDtypeStruct((1, 1), jnp.float32),
        scratch_shapes=[pltpu.VMEM((_NBLK, _ROWS), jnp.float32)] * 4,
    )(bvec, epv, Wa.T, Wb.T, z, lab3, w3)
    return out[0, 0]
